# baseline (speedup 1.0000x reference)
import functools

import jax
import jax.numpy as jnp
from jax import lax
from jax.experimental import pallas as pl
from jax.experimental.pallas import tpu as pltpu
from jax.experimental.pallas import tpu_sc as plsc

B = 4096
L = 200
DIM = 32

_info = plsc.get_sparse_core_info()
_NC, _NS = _info.num_cores, _info.num_subcores
NW = _NC * _NS
TOTAL = B * L
ROWS_PER_W = TOTAL // NW
CHUNK = 128
NCHUNK = ROWS_PER_W // CHUNK
CTX_PER_W = B // NW


def _sc_gather(qid, did, context_table, example_table):
  mesh = plsc.VectorSubcoreMesh(core_axis_name="c", subcore_axis_name="s")

  @functools.partial(
      pl.kernel,
      mesh=mesh,
      out_type=(jax.ShapeDtypeStruct((B, DIM), jnp.float32),
                jax.ShapeDtypeStruct((TOTAL, DIM), jnp.float32)),
      scratch_types=[
          pltpu.VMEM((NCHUNK, CHUNK), jnp.int32),
          pltpu.VMEM((CHUNK, DIM), jnp.float32),
          pltpu.VMEM((CTX_PER_W,), jnp.int32),
          pltpu.VMEM((CTX_PER_W, DIM), jnp.float32),
          pltpu.SemaphoreType.DMA,
      ],
      compiler_params=pltpu.CompilerParams(use_tc_tiling_on_sc=False),
  )
  def k(qid_hbm, did_hbm, ctx_tab, ex_tab, ctx_out, ex_out,
        idx_v, rows_v, cidx_v, crows_v, sem):
    wid = lax.axis_index("s") * _NC + lax.axis_index("c")

    pltpu.sync_copy(qid_hbm.at[wid], cidx_v)
    pltpu.async_copy(ctx_tab.at[cidx_v], crows_v, sem).wait()
    pltpu.sync_copy(crows_v, ctx_out.at[pl.ds(wid * CTX_PER_W, CTX_PER_W)])

    pltpu.sync_copy(did_hbm.at[wid], idx_v)
    base = wid * ROWS_PER_W

    def step(j, carry):
      pltpu.async_copy(ex_tab.at[idx_v.at[j]], rows_v, sem).wait()
      pltpu.sync_copy(rows_v, ex_out.at[pl.ds(base + j * CHUNK, CHUNK)])
      return carry

    lax.fori_loop(0, NCHUNK, step, 0)

  return k(qid, did, context_table, example_table)


def kernel(query_id, doc_id, context_table, example_table):
  qid = query_id.astype(jnp.int32).reshape(NW, CTX_PER_W)
  did = doc_id.astype(jnp.int32).reshape(NW, NCHUNK, CHUNK)
  ctx_emb, ex_flat = _sc_gather(qid, did, context_table, example_table)
  return (ctx_emb, ex_flat.reshape(B, L, DIM))


# speedup vs baseline: 1.1065x; 1.1065x over previous
import functools

import jax
import jax.numpy as jnp
from jax import lax
from jax.experimental import pallas as pl
from jax.experimental.pallas import tpu as pltpu
from jax.experimental.pallas import tpu_sc as plsc

B = 4096
L = 200
DIM = 32

_info = plsc.get_sparse_core_info()
_NC, _NS = _info.num_cores, _info.num_subcores
NW = _NC * _NS
TOTAL = B * L
ROWS_PER_W = TOTAL // NW
CHUNK = 128
NCHUNK = ROWS_PER_W // CHUNK
CTX_PER_W = B // NW
K = 8
D = 4


def _sc_gather(qid, did, context_table, example_table):
  mesh = plsc.VectorSubcoreMesh(core_axis_name="c", subcore_axis_name="s")

  @functools.partial(
      pl.kernel,
      mesh=mesh,
      out_type=(jax.ShapeDtypeStruct((B, DIM), jnp.float32),
                jax.ShapeDtypeStruct((TOTAL, DIM), jnp.float32)),
      scratch_types=[
          pltpu.VMEM((NCHUNK, CHUNK), jnp.int32),
          pltpu.VMEM((K, CHUNK, DIM), jnp.float32),
          pltpu.VMEM((CTX_PER_W,), jnp.int32),
          pltpu.VMEM((CTX_PER_W, DIM), jnp.float32),
          pltpu.SemaphoreType.DMA((K,)),
          pltpu.SemaphoreType.DMA((K,)),
          pltpu.SemaphoreType.DMA,
      ],
      compiler_params=pltpu.CompilerParams(use_tc_tiling_on_sc=False),
  )
  def k(qid_hbm, did_hbm, ctx_tab, ex_tab, ctx_out, ex_out,
        idx_v, rows_v, cidx_v, crows_v, gsem, wsem, csem):
    wid = lax.axis_index("s") * _NC + lax.axis_index("c")
    base = wid * ROWS_PER_W

    def start_gather(j, b):
      pltpu.async_copy(ex_tab.at[idx_v.at[j]], rows_v.at[b], gsem.at[b])

    def wait_gather(j, b):
      pltpu.make_async_copy(
          ex_tab.at[idx_v.at[j]], rows_v.at[b], gsem.at[b]).wait()

    def start_write(j, b):
      pltpu.async_copy(
          rows_v.at[b], ex_out.at[pl.ds(base + j * CHUNK, CHUNK)], wsem.at[b])

    def wait_write(j, b):
      pltpu.make_async_copy(
          rows_v.at[b], ex_out.at[pl.ds(base + j * CHUNK, CHUNK)],
          wsem.at[b]).wait()

    pltpu.sync_copy(did_hbm.at[wid], idx_v)
    pltpu.sync_copy(qid_hbm.at[wid], cidx_v)
    pltpu.async_copy(ctx_tab.at[cidx_v], crows_v, csem).wait()
    pltpu.sync_copy(crows_v, ctx_out.at[pl.ds(wid * CTX_PER_W, CTX_PER_W)])

    for j in range(D):
      start_gather(j, j % K)

    for j in range(K - D):
      wait_gather(j, j % K)
      start_write(j, j % K)
      start_gather(j + D, (j + D) % K)

    def group(g, carry):
      for b in range(K):
        j = (K - D) + g * K + b
        bj = (K - D + b) % K
        bf = (b + K - D + D) % K
        wait_gather(j, bj)
        start_write(j, bj)
        wait_write(j + D - K, bf)
        start_gather(j + D, bf)
      return carry

    lax.fori_loop(0, (NCHUNK - K) // K, group, 0)

    for j in range(NCHUNK - D, NCHUNK):
      wait_gather(j, j % K)
      start_write(j, j % K)

    for j in range(NCHUNK - K, NCHUNK):
      wait_write(j, j % K)

  return k(qid, did, context_table, example_table)


def kernel(query_id, doc_id, context_table, example_table):
  qid = query_id.astype(jnp.int32).reshape(NW, CTX_PER_W)
  did = doc_id.astype(jnp.int32).reshape(NW, NCHUNK, CHUNK)
  ctx_emb, ex_flat = _sc_gather(qid, did, context_table, example_table)
  return (ctx_emb, ex_flat.reshape(B, L, DIM))


# speedup vs baseline: 1.1374x; 1.0279x over previous
import functools

import jax
import jax.numpy as jnp
from jax import lax
from jax.experimental import pallas as pl
from jax.experimental.pallas import tpu as pltpu
from jax.experimental.pallas import tpu_sc as plsc

B = 4096
L = 200
DIM = 32
V = 1000000
NLINE = V // 4

_info = plsc.get_sparse_core_info()
_NC, _NS = _info.num_cores, _info.num_subcores
NW = _NC * _NS
TOTAL = B * L
ROWS_PER_W = TOTAL // NW
CHUNK = 128
NCHUNK = ROWS_PER_W // CHUNK
CTX_PER_W = B // NW

RP_ROWS = 320
RP_LINES = RP_ROWS // 4
RP_NCH = V // RP_ROWS
RP_SLOTS = -(-RP_NCH // NW)


def _repack_and_context(qid, context_table, example_table):
  mesh = plsc.VectorSubcoreMesh(core_axis_name="c", subcore_axis_name="s")

  @functools.partial(
      pl.kernel,
      mesh=mesh,
      out_type=(jax.ShapeDtypeStruct((NLINE, 128), jnp.float32),
                jax.ShapeDtypeStruct((B, DIM), jnp.float32)),
      scratch_types=[
          pltpu.VMEM((2, RP_ROWS, DIM), jnp.float32),
          pltpu.VMEM((2, RP_LINES, 128), jnp.float32),
          pltpu.VMEM((CTX_PER_W,), jnp.int32),
          pltpu.VMEM((CTX_PER_W, DIM), jnp.float32),
          pltpu.SemaphoreType.DMA((2,)),
          pltpu.SemaphoreType.DMA,
      ],
  )
  def k(qid_hbm, ctx_tab, ex_tab, scr, ctx_out, buf, pbuf, cidx_v, crows_v,
        rsem, csem):
    wid = lax.axis_index("s") * _NC + lax.axis_index("c")

    pltpu.sync_copy(qid_hbm.at[wid], cidx_v)
    def cgroup(g, carry):
      vec = cidx_v[pl.ds(g * 16, 16)]
      for i in range(16):
        pltpu.async_copy(ctx_tab.at[vec[i]], crows_v.at[g * 16 + i], csem)
      for i in range(16):
        pltpu.make_async_copy(
            ctx_tab.at[vec[i]], crows_v.at[g * 16 + i], csem).wait()
      return carry
    lax.fori_loop(0, CTX_PER_W // 16, cgroup, 0)
    pltpu.sync_copy(crows_v, ctx_out.at[pl.ds(wid * CTX_PER_W, CTX_PER_W)])

    def start_read(c, b):
      pltpu.async_copy(
          ex_tab.at[pl.ds(c * RP_ROWS, RP_ROWS)], buf.at[b], rsem.at[b])

    def wait_read(c, b):
      pltpu.make_async_copy(
          ex_tab.at[pl.ds(c * RP_ROWS, RP_ROWS)], buf.at[b],
          rsem.at[b]).wait()

    def pack(b):
      def line(i, carry):
        for rr in range(4):
          for h in range(2):
            v = buf[b, i * 4 + rr, pl.ds(h * 16, 16)]
            pbuf[b, i, pl.ds(rr * 32 + h * 16, 16)] = v
        return carry
      lax.fori_loop(0, RP_LINES, line, 0)

    def write(c, b):
      pltpu.sync_copy(pbuf.at[b], scr.at[pl.ds(c * RP_LINES, RP_LINES)])

    start_read(wid, 0)
    def group(g, carry):
      for s in range(2):
        j = g * 2 + s
        c = wid + j * NW
        @pl.when(c < RP_NCH)
        def _():
          nxt = c + NW
          @pl.when(nxt < RP_NCH)
          def _():
            start_read(nxt, 1 - s)
          wait_read(c, s)
          pack(s)
          write(c, s)
      return carry
    lax.fori_loop(0, RP_SLOTS // 2, group, 0)

  return k(qid, context_table, example_table)


def _gather(did, scr):
  mesh = plsc.VectorSubcoreMesh(core_axis_name="c", subcore_axis_name="s")

  @functools.partial(
      pl.kernel,
      mesh=mesh,
      out_type=jax.ShapeDtypeStruct((TOTAL, DIM), jnp.float32),
      scratch_types=[
          pltpu.VMEM((NCHUNK, CHUNK), jnp.int32),
          pltpu.VMEM((2, CHUNK), jnp.int32),
          pltpu.VMEM((2, CHUNK, 128), jnp.float32),
          pltpu.VMEM((2, CHUNK, DIM), jnp.float32),
          pltpu.SemaphoreType.DMA((2,)),
      ],
  )
  def k(did_hbm, scr_hbm, ex_out, idx_v, lbuf, rows_v, cbuf, gsem):
    wid = lax.axis_index("s") * _NC + lax.axis_index("c")
    base = wid * ROWS_PER_W

    pltpu.sync_copy(did_hbm.at[wid], idx_v)

    def compute_lines(j, b):
      for g in range(CHUNK // 16):
        rvec = idx_v[j, pl.ds(g * 16, 16)]
        lbuf[b, pl.ds(g * 16, 16)] = lax.shift_right_logical(rvec, 2)

    def start_gather(b):
      pltpu.async_copy(scr_hbm.at[lbuf.at[b]], rows_v.at[b], gsem.at[b])

    def wait_gather(b):
      pltpu.make_async_copy(
          scr_hbm.at[lbuf.at[b]], rows_v.at[b], gsem.at[b]).wait()

    def extract(j, b):
      def grp(g, carry):
        rvec = idx_v[j, pl.ds(g * 16, 16)]
        qvec = lax.shift_left(
            lax.bitwise_and(rvec, jnp.int32(3)), jnp.int32(5))
        for i in range(16):
          off = qvec[i]
          v0 = rows_v[b, g * 16 + i, pl.ds(off, 16)]
          v1 = rows_v[b, g * 16 + i, pl.ds(off + 16, 16)]
          cbuf[b, g * 16 + i, pl.ds(0, 16)] = v0
          cbuf[b, g * 16 + i, pl.ds(16, 16)] = v1
        return carry
      lax.fori_loop(0, CHUNK // 16, grp, 0)

    def write(j, b):
      pltpu.sync_copy(
          cbuf.at[b], ex_out.at[pl.ds(base + j * CHUNK, CHUNK)])

    compute_lines(0, 0)
    start_gather(0)
    def group(g, carry):
      for s in range(2):
        j = g * 2 + s
        @pl.when(j + 1 < NCHUNK)
        def _():
          compute_lines(j + 1, 1 - s)
          start_gather(1 - s)
        wait_gather(s)
        extract(j, s)
        write(j, s)
      return carry
    lax.fori_loop(0, NCHUNK // 2, group, 0)

  return k(did, scr)


def kernel(query_id, doc_id, context_table, example_table):
  qid = query_id.astype(jnp.int32).reshape(NW, CTX_PER_W)
  did = doc_id.astype(jnp.int32).reshape(NW, NCHUNK, CHUNK)
  scr, ctx_emb = _repack_and_context(qid, context_table, example_table)
  ex_flat = _gather(did, scr)
  return (ctx_emb, ex_flat.reshape(B, L, DIM))
